# initial kernel scaffold (unmeasured)
import jax
import jax.numpy as jnp
from jax import lax
from jax.experimental import pallas as pl
from jax.experimental.pallas import tpu as pltpu

N_DEV = 16
N_TOK = 2048
D_IN = 512
D_OUT = 1024
N_EXP = 128
E_LOCAL = N_EXP // N_DEV
CHUNK = N_TOK // N_DEV


def kernel(x, router_W, route_idx, expert_W, shared_W):
    def body(x_ref, rw_ref, ri_ref, ew_ref, sw_ref, out_ref,
             comm_ref, send_sems, recv_sems):
        my_p = lax.axis_index("i")
        left = jnp.mod(my_p - 1, N_DEV)
        right = jnp.mod(my_p + 1, N_DEV)

        barrier_sem = pltpu.get_barrier_semaphore()
        for nbr in (left, right):
            pl.semaphore_signal(
                barrier_sem, inc=1,
                device_id=(nbr,), device_id_type=pl.DeviceIdType.MESH,
            )
        pl.semaphore_wait(barrier_sem, 2)

        xv = x_ref[...]
        scores = jnp.dot(xv, rw_ref[...], preferred_element_type=jnp.float32)
        smax = jnp.max(scores, axis=1, keepdims=True)
        sexp = jnp.exp(scores - smax)
        ssum = jnp.sum(sexp, axis=1, keepdims=True)
        rid = ri_ref[...]
        onehot = lax.broadcasted_iota(jnp.int32, (N_TOK, N_EXP), 1) == rid
        sel = jnp.sum(jnp.where(onehot, sexp, 0.0), axis=1, keepdims=True)
        coeff = sel / ssum

        partial = jnp.dot(xv, sw_ref[...],
                          preferred_element_type=jnp.float32) * (1.0 / N_DEV)
        for e in range(E_LOCAL):
            ge = my_p * E_LOCAL + e
            w = jnp.where(rid == ge, coeff, 0.0)
            partial = partial + jnp.dot(
                xv * w, ew_ref[e, :, :], preferred_element_type=jnp.float32)
        out_ref[...] = partial

        for s in range(N_DEV - 1):
            send_row = jnp.mod(my_p - s, N_DEV) * CHUNK
            rdma = pltpu.make_async_remote_copy(
                src_ref=out_ref.at[pl.ds(send_row, CHUNK), :],
                dst_ref=comm_ref.at[s],
                send_sem=send_sems.at[s],
                recv_sem=recv_sems.at[s],
                device_id=(right,),
                device_id_type=pl.DeviceIdType.MESH,
            )
            rdma.start()
            rdma.wait()
            acc_row = jnp.mod(my_p - s - 1, N_DEV) * CHUNK
            out_ref[pl.ds(acc_row, CHUNK), :] = (
                out_ref[pl.ds(acc_row, CHUNK), :] + comm_ref[s])

        for s in range(N_DEV - 1):
            row = jnp.mod(my_p + 1 - s, N_DEV) * CHUNK
            rdma = pltpu.make_async_remote_copy(
                src_ref=out_ref.at[pl.ds(row, CHUNK), :],
                dst_ref=out_ref.at[pl.ds(row, CHUNK), :],
                send_sem=send_sems.at[N_DEV - 1 + s],
                recv_sem=recv_sems.at[N_DEV - 1 + s],
                device_id=(right,),
                device_id_type=pl.DeviceIdType.MESH,
            )
            rdma.start()
            rdma.wait()

    return pl.pallas_call(
        body,
        out_shape=jax.ShapeDtypeStruct((N_TOK, D_OUT), jnp.float32),
        in_specs=[
            pl.BlockSpec(memory_space=pltpu.VMEM),
            pl.BlockSpec(memory_space=pltpu.VMEM),
            pl.BlockSpec(memory_space=pltpu.VMEM),
            pl.BlockSpec(memory_space=pltpu.VMEM),
            pl.BlockSpec(memory_space=pltpu.VMEM),
        ],
        out_specs=pl.BlockSpec(memory_space=pltpu.VMEM),
        scratch_shapes=[
            pltpu.VMEM((N_DEV - 1, CHUNK, D_OUT), jnp.float32),
            pltpu.SemaphoreType.DMA((2 * (N_DEV - 1),)),
            pltpu.SemaphoreType.DMA((2 * (N_DEV - 1),)),
        ],
        compiler_params=pltpu.CompilerParams(collective_id=0),
    )(x, router_W, route_idx, expert_W, shared_W)


# baseline (device time: 268739 ns/iter reference)
import jax
import jax.numpy as jnp
from jax import lax
from jax.experimental import pallas as pl
from jax.experimental.pallas import tpu as pltpu

N_DEV = 16
N_TOK = 2048
D_IN = 512
D_OUT = 1024
N_EXP = 128
E_LOCAL = N_EXP // N_DEV
CHUNK = N_TOK // N_DEV


def kernel(x, router_W, route_idx, expert_W, shared_W):
    def body(x_ref, rw_ref, ri_ref, ew_ref, sw_ref, out_ref,
             comm_ref, send_sems, recv_sems):
        my_p = lax.axis_index("i")
        left = jnp.mod(my_p - 1, N_DEV)
        right = jnp.mod(my_p + 1, N_DEV)

        barrier_sem = pltpu.get_barrier_semaphore()
        for nbr in (left, right):
            pl.semaphore_signal(
                barrier_sem, inc=1,
                device_id=(nbr,), device_id_type=pl.DeviceIdType.MESH,
            )
        pl.semaphore_wait(barrier_sem, 2)

        xv = x_ref[...]
        scores = jnp.dot(xv, rw_ref[...], preferred_element_type=jnp.float32)
        smax = jnp.max(scores, axis=1, keepdims=True)
        sexp = jnp.exp(scores - smax)
        ssum = jnp.sum(sexp, axis=1, keepdims=True)
        rid = ri_ref[...]
        onehot = lax.broadcasted_iota(jnp.int32, (N_TOK, N_EXP), 1) == rid
        sel = jnp.sum(jnp.where(onehot, sexp, 0.0), axis=1, keepdims=True)
        coeff = sel / ssum

        partial = jnp.dot(xv, sw_ref[...],
                          preferred_element_type=jnp.float32) * (1.0 / N_DEV)
        for e in range(E_LOCAL):
            ge = my_p * E_LOCAL + e
            w = jnp.where(rid == ge, coeff, 0.0)
            partial = partial + jnp.dot(
                xv * w, ew_ref[e, :, :], preferred_element_type=jnp.float32)
        out_ref[...] = partial

        for s in range(N_DEV - 1):
            send_row = jnp.mod(my_p - s, N_DEV) * CHUNK
            rdma = pltpu.make_async_remote_copy(
                src_ref=out_ref.at[pl.ds(send_row, CHUNK), :],
                dst_ref=comm_ref.at[s],
                send_sem=send_sems.at[s],
                recv_sem=recv_sems.at[s],
                device_id=(right,),
                device_id_type=pl.DeviceIdType.MESH,
            )
            rdma.start()
            rdma.wait()
            acc_row = jnp.mod(my_p - s - 1, N_DEV) * CHUNK
            out_ref[pl.ds(acc_row, CHUNK), :] = (
                out_ref[pl.ds(acc_row, CHUNK), :] + comm_ref[s])

        for s in range(N_DEV - 1):
            row = jnp.mod(my_p + 1 - s, N_DEV) * CHUNK
            rdma = pltpu.make_async_remote_copy(
                src_ref=out_ref.at[pl.ds(row, CHUNK), :],
                dst_ref=out_ref.at[pl.ds(row, CHUNK), :],
                send_sem=send_sems.at[N_DEV - 1 + s],
                recv_sem=recv_sems.at[N_DEV - 1 + s],
                device_id=(right,),
                device_id_type=pl.DeviceIdType.MESH,
            )
            rdma.start()
            rdma.wait()

    return pl.pallas_call(
        body,
        out_shape=jax.ShapeDtypeStruct((N_TOK, D_OUT), jnp.float32),
        in_specs=[
            pl.BlockSpec(memory_space=pltpu.VMEM),
            pl.BlockSpec(memory_space=pltpu.VMEM),
            pl.BlockSpec(memory_space=pltpu.VMEM),
            pl.BlockSpec(memory_space=pltpu.VMEM),
            pl.BlockSpec(memory_space=pltpu.VMEM),
        ],
        out_specs=pl.BlockSpec(memory_space=pltpu.VMEM),
        scratch_shapes=[
            pltpu.VMEM((N_DEV - 1, CHUNK, D_OUT), jnp.float32),
            pltpu.SemaphoreType.DMA((2 * (N_DEV - 1),)),
            pltpu.SemaphoreType.DMA((2 * (N_DEV - 1),)),
        ],
        compiler_params=pltpu.CompilerParams(
            collective_id=0, vmem_limit_bytes=100 * 1024 * 1024),
    )(x, router_W, route_idx, expert_W, shared_W)


# device time: 84429 ns/iter; 3.1830x vs baseline; 3.1830x over previous
import jax
import jax.numpy as jnp
from jax import lax
from jax.experimental import pallas as pl
from jax.experimental.pallas import tpu as pltpu

N_DEV = 16
N_TOK = 2048
D_IN = 512
D_OUT = 1024
HALF = D_OUT // 2
N_EXP = 128
E_LOCAL = N_EXP // N_DEV
CHUNK = N_TOK // N_DEV
HOPS = N_DEV - 1

RING = [0, 1, 5, 9, 13, 14, 10, 6, 2, 3, 7, 11, 15, 12, 8, 4]
INV = [0] * N_DEV
for _q, _k in enumerate(RING):
    INV[_k] = _q
RIGHT_OF = [RING[(INV[k] + 1) % N_DEV] for k in range(N_DEV)]
LEFT_OF = [RING[(INV[k] - 1) % N_DEV] for k in range(N_DEV)]

AG_HOPS = N_DEV // 2
_READY_AT: dict[int, list[int]] = {h: [] for h in range(AG_HOPS)}
_READY_AT[0] = [0]
for _j in range(1, 8):
    _READY_AT[_j] = [_j, -_j]
_READY_AT[6].append(8)


def kernel(x, router_W, route_idx, expert_W, shared_W):
    def body(x_ref, rw_ref, ri_ref, ew_ref, sw_ref, out_ref,
             ring_buf, comm, coeff_buf, rs_send, rs_recv,
             ag_send, ag_recv):
        my_k = lax.axis_index("i")

        def lut(table):
            v = jnp.int32(table[0])
            for i in range(1, N_DEV):
                v = jnp.where(my_k == i, jnp.int32(table[i]), v)
            return v

        my_p = lut(INV)
        right = lut(RIGHT_OF)
        left = lut(LEFT_OF)

        scores = jnp.dot(x_ref[...], rw_ref[...],
                         preferred_element_type=jnp.float32)
        smax = jnp.max(scores, axis=1, keepdims=True)
        sexp = jnp.exp(scores - smax)
        ssum = jnp.sum(sexp, axis=1, keepdims=True)
        onehot = (lax.broadcasted_iota(jnp.int32, (N_TOK, N_EXP), 1)
                  == ri_ref[...])
        sel = jnp.sum(jnp.where(onehot, sexp, 0.0), axis=1, keepdims=True)
        coeff_buf[...] = sel / ssum

        barrier_sem = pltpu.get_barrier_semaphore()
        for nbr in (left, right):
            pl.semaphore_signal(
                barrier_sem, inc=1,
                device_id=(nbr,), device_id_type=pl.DeviceIdType.MESH,
            )
        pl.semaphore_wait(barrier_sem, 2)

        def compute_half(c, col0):
            row = c * CHUNK
            xc = x_ref[pl.ds(row, CHUNK), :]
            ric = ri_ref[pl.ds(row, CHUNK), :]
            coeff = coeff_buf[pl.ds(row, CHUNK), :]
            part = jnp.zeros((CHUNK, HALF), jnp.float32)
            for e in range(E_LOCAL):
                ge = my_k * E_LOCAL + e
                w = jnp.where(ric == ge, coeff, 0.0)
                part = part + jnp.dot(
                    xc * w, ew_ref[e, :, pl.ds(col0, HALF)],
                    preferred_element_type=jnp.float32)
            ring_buf[pl.ds(row, CHUNK), pl.ds(col0, HALF)] = (
                part.astype(jnp.bfloat16))

        def combine_chunk(c):
            row = c * CHUNK
            xc = x_ref[pl.ds(row, CHUNK), :]
            shared = jnp.dot(xc, sw_ref[...],
                             preferred_element_type=jnp.float32)
            out_ref[pl.ds(row, CHUNK), :] = (
                ring_buf[pl.ds(row, CHUNK), :].astype(jnp.float32) + shared)

        def mk_rs(chain, h, c, col0, dev):
            rc = jnp.mod(c, N_DEV) * CHUNK
            return pltpu.make_async_remote_copy(
                src_ref=ring_buf.at[pl.ds(rc, CHUNK), pl.ds(col0, HALF)],
                dst_ref=comm.at[chain, h],
                send_sem=rs_send.at[chain, h],
                recv_sem=rs_recv.at[chain, h],
                device_id=(dev,), device_id_type=pl.DeviceIdType.MESH,
            )

        def rs_ar(h):
            return mk_rs(0, h, my_p + 9 - h, 0, right)

        def rs_al(h):
            return mk_rs(1, h, my_p - 6 + h, 0, left)

        def rs_bl(h):
            return mk_rs(2, h, my_p + 7 + h, HALF, left)

        def rs_br(h):
            return mk_rs(3, h, my_p + 6 - h, HALF, right)

        def acc(chain, h, c, col0):
            rc = jnp.mod(c, N_DEV) * CHUNK
            ring_buf[pl.ds(rc, CHUNK), pl.ds(col0, HALF)] = (
                ring_buf[pl.ds(rc, CHUNK), pl.ds(col0, HALF)]
                + comm[chain, h])

        compute_half(jnp.mod(my_p - 7, N_DEV), 0)
        p_ar = [rs_ar(0)]
        p_ar[0].start()
        compute_half(jnp.mod(my_p + 7, N_DEV), HALF)
        p_bl = [rs_bl(0)]
        p_bl[0].start()
        compute_half(jnp.mod(my_p - 6, N_DEV), 0)
        p_al = [rs_al(0)]
        p_al[0].start()
        compute_half(jnp.mod(my_p + 6, N_DEV), HALF)
        p_br = [rs_br(0)]
        p_br[0].start()
        for h in range(AG_HOPS):
            if h < AG_HOPS - 1:
                compute_half(jnp.mod(my_p + 8 - h, N_DEV), 0)
                compute_half(jnp.mod(my_p - 5 + h, N_DEV), 0)
                compute_half(jnp.mod(my_p - 8 + h, N_DEV), HALF)
                compute_half(jnp.mod(my_p + 5 - h, N_DEV), HALF)
            p_ar[h].wait_recv()
            acc(0, h, my_p + 8 - h, 0)
            if h < AG_HOPS - 1:
                d = rs_ar(h + 1)
                d.start()
                p_ar.append(d)
            p_bl[h].wait_recv()
            acc(2, h, my_p - 8 + h, HALF)
            if h < AG_HOPS - 1:
                d = rs_bl(h + 1)
                d.start()
                p_bl.append(d)
            if h < AG_HOPS - 1:
                p_al[h].wait_recv()
                acc(1, h, my_p - 5 + h, 0)
                if h < AG_HOPS - 2:
                    d = rs_al(h + 1)
                    d.start()
                    p_al.append(d)
                p_br[h].wait_recv()
                acc(3, h, my_p + 5 - h, HALF)
                if h < AG_HOPS - 2:
                    d = rs_br(h + 1)
                    d.start()
                    p_br.append(d)

        def mk_ag(chain, h, c, col0, dev):
            rc = jnp.mod(c, N_DEV) * CHUNK
            return pltpu.make_async_remote_copy(
                src_ref=ring_buf.at[pl.ds(rc, CHUNK), pl.ds(col0, HALF)],
                dst_ref=ring_buf.at[pl.ds(rc, CHUNK), pl.ds(col0, HALF)],
                send_sem=ag_send.at[chain, h],
                recv_sem=ag_recv.at[chain, h],
                device_id=(dev,), device_id_type=pl.DeviceIdType.MESH,
            )

        def ag_ar(h):
            return mk_ag(0, h, my_p + 1 - h, 0, right)

        def ag_al(h):
            return mk_ag(1, h, my_p + 1 + h, 0, left)

        def ag_bl(h):
            return mk_ag(2, h, my_p - 1 + h, HALF, left)

        def ag_br(h):
            return mk_ag(3, h, my_p - 1 - h, HALF, right)

        ar = [ag_ar(0)]
        al = [ag_al(0)]
        bl = [ag_bl(0)]
        br = [ag_br(0)]
        for d in (ar[0], al[0], bl[0], br[0]):
            d.start()
        for h in range(AG_HOPS):
            ar[h].wait_recv()
            if h < AG_HOPS - 1:
                d = ag_ar(h + 1)
                d.start()
                ar.append(d)
            bl[h].wait_recv()
            if h < AG_HOPS - 1:
                d = ag_bl(h + 1)
                d.start()
                bl.append(d)
            if h < AG_HOPS - 1:
                al[h].wait_recv()
                if h < AG_HOPS - 2:
                    d = ag_al(h + 1)
                    d.start()
                    al.append(d)
                br[h].wait_recv()
                if h < AG_HOPS - 2:
                    d = ag_br(h + 1)
                    d.start()
                    br.append(d)
            for off in _READY_AT[h]:
                combine_chunk(jnp.mod(my_p + off, N_DEV))
        for d in p_ar + p_al + p_bl + p_br + ar + al + bl + br:
            d.wait_send()

    return pl.pallas_call(
        body,
        out_shape=jax.ShapeDtypeStruct((N_TOK, D_OUT), jnp.float32),
        in_specs=[
            pl.BlockSpec(memory_space=pltpu.VMEM),
            pl.BlockSpec(memory_space=pltpu.VMEM),
            pl.BlockSpec(memory_space=pltpu.VMEM),
            pl.BlockSpec(memory_space=pltpu.VMEM),
            pl.BlockSpec(memory_space=pltpu.VMEM),
        ],
        out_specs=pl.BlockSpec(memory_space=pltpu.VMEM),
        scratch_shapes=[
            pltpu.VMEM((N_TOK, D_OUT), jnp.bfloat16),
            pltpu.VMEM((4, AG_HOPS, CHUNK, HALF), jnp.bfloat16),
            pltpu.VMEM((N_TOK, 1), jnp.float32),
            pltpu.SemaphoreType.DMA((4, AG_HOPS)),
            pltpu.SemaphoreType.DMA((4, AG_HOPS)),
            pltpu.SemaphoreType.DMA((4, AG_HOPS)),
            pltpu.SemaphoreType.DMA((4, AG_HOPS)),
        ],
        compiler_params=pltpu.CompilerParams(
            collective_id=0, vmem_limit_bytes=100 * 1024 * 1024),
    )(x, router_W, route_idx, expert_W, shared_W)
